# baseline (speedup 1.0000x reference)
import functools

import jax
import jax.numpy as jnp
from jax import lax
from jax.experimental import pallas as pl
from jax.experimental.pallas import tpu as pltpu
from jax.experimental.pallas import tpu_sc as plsc

B, C_IN, C_OUT, N, K = 4, 64, 128, 2048, 16
TN = 256
NEG_BIG = 3.0e38

NC, NS, L = 2, 16, 16
NW = NC * NS
PPW = (B * N) // NW
PC = 32
NCH = PPW // PC
ROWS = PC * K


def _knn_idx_body(rows_ref, full_ref, idx_ref):
    b = pl.program_id(0)
    rows = rows_ref[0]
    full = full_ref[0]
    rn_rows = jnp.sum(rows * rows, axis=1, keepdims=True)
    rn_all = jnp.sum(full * full, axis=1)[None, :]
    cross = lax.dot_general(
        rows, full, dimension_numbers=(((1,), (1,)), ((), ())),
        preferred_element_type=jnp.float32)
    d = rn_rows + rn_all - 2.0 * cross

    iota = lax.broadcasted_iota(jnp.int32, (TN, N), 1)
    cols = []
    for _ in range(K):
        v = jnp.min(d, axis=1, keepdims=True)
        eq = d == v
        idxj = jnp.min(jnp.where(eq, iota, N), axis=1)
        cols.append(idxj)
        d = jnp.where(eq, NEG_BIG, d)
    idx_ref[...] = jnp.stack(cols, axis=1) + b * N


def _gather_max_body(idx_hbm, xt_hbm, out_hbm, idx_v, rows_v, xm_v, sem0, sem1):
    wid = lax.axis_index("s") * NC + lax.axis_index("c")
    base = wid * PPW
    pltpu.sync_copy(idx_hbm.at[pl.ds(base * K, PPW * K)], idx_v)

    sems = (sem0, sem1)
    pending = pltpu.async_copy(
        xt_hbm.at[idx_v.at[pl.ds(0, ROWS)]], rows_v.at[0], sems[0])
    for c in range(NCH):
        cur = pending
        if c + 1 < NCH:
            pending = pltpu.async_copy(
                xt_hbm.at[idx_v.at[pl.ds((c + 1) * ROWS, ROWS)]],
                rows_v.at[(c + 1) % 2], sems[(c + 1) % 2])
        cur.wait()
        buf = c % 2

        def point_body(p, carry, _c=c, _buf=buf):
            for ch in range(C_IN // L):
                sl = pl.ds(ch * L, L)
                acc = rows_v[_buf, p * K, sl]
                for j in range(1, K):
                    acc = jnp.maximum(acc, rows_v[_buf, p * K + j, sl])
                xm_v[_c * PC + p, sl] = acc
            return carry

        lax.fori_loop(0, PC, point_body, 0)

    pltpu.sync_copy(xm_v, out_hbm.at[pl.ds(base, PPW)])


_gather_max = functools.partial(
    pl.kernel,
    out_type=jax.ShapeDtypeStruct((B * N, C_IN), jnp.float32),
    mesh=plsc.VectorSubcoreMesh(core_axis_name="c", subcore_axis_name="s"),
    compiler_params=pltpu.CompilerParams(use_tc_tiling_on_sc=False),
    scratch_types=[
        pltpu.VMEM((PPW * K,), jnp.int32),
        pltpu.VMEM((2, ROWS, C_IN), jnp.float32),
        pltpu.VMEM((PPW, C_IN), jnp.float32),
        pltpu.SemaphoreType.DMA,
        pltpu.SemaphoreType.DMA,
    ],
)(_gather_max_body)


def _conv_stats_body(xm_ref, w_ref, b_ref, y_ref, sums_ref):
    step = pl.program_id(0) * pl.num_programs(1) + pl.program_id(1)
    xm = xm_ref[0]
    w = w_ref[...]
    y = lax.dot_general(
        xm, w, dimension_numbers=(((1,), (1,)), ((), ())),
        preferred_element_type=jnp.float32) + b_ref[...]
    y_ref[0] = y

    @pl.when(step == 0)
    def _():
        sums_ref[...] = jnp.zeros_like(sums_ref)

    sums_ref[0:1, :] += jnp.sum(y, axis=0, keepdims=True)
    sums_ref[1:2, :] += jnp.sum(y * y, axis=0, keepdims=True)


def _bn_relu_body(y_ref, sums_ref, g_ref, bt_ref, out_ref):
    y = y_ref[0]
    cnt = float(B * N)
    mean = sums_ref[0:1, :] / cnt
    var = sums_ref[1:2, :] / cnt - mean * mean
    scale = g_ref[...] / jnp.sqrt(var + 1e-5)
    shift = bt_ref[...] - mean * scale
    r = jnp.maximum(y * scale + shift, 0.0)
    out_ref[0] = r.T


def kernel(x, conv_w, conv_b, bn_gamma, bn_beta):
    xt = jnp.transpose(x, (0, 2, 1))
    w = conv_w[:, :, 0]

    idx = pl.pallas_call(
        _knn_idx_body,
        grid=(B, N // TN),
        in_specs=[
            pl.BlockSpec((1, TN, C_IN), lambda b, i: (b, i, 0)),
            pl.BlockSpec((1, N, C_IN), lambda b, i: (b, 0, 0)),
        ],
        out_specs=pl.BlockSpec((TN, K), lambda b, i: (b * (N // TN) + i, 0)),
        out_shape=jax.ShapeDtypeStruct((B * N, K), jnp.int32),
    )(xt, xt)

    xm = _gather_max(idx.reshape(B * N * K), xt.reshape(B * N, C_IN))
    xm = xm.reshape(B, N, C_IN)

    y, sums = pl.pallas_call(
        _conv_stats_body,
        grid=(B, N // TN),
        in_specs=[
            pl.BlockSpec((1, TN, C_IN), lambda b, i: (b, i, 0)),
            pl.BlockSpec((C_OUT, C_IN), lambda b, i: (0, 0)),
            pl.BlockSpec((1, C_OUT), lambda b, i: (0, 0)),
        ],
        out_specs=[
            pl.BlockSpec((1, TN, C_OUT), lambda b, i: (b, i, 0)),
            pl.BlockSpec((8, C_OUT), lambda b, i: (0, 0)),
        ],
        out_shape=[
            jax.ShapeDtypeStruct((B, N, C_OUT), jnp.float32),
            jax.ShapeDtypeStruct((8, C_OUT), jnp.float32),
        ],
    )(xm, w, conv_b[None, :])

    out = pl.pallas_call(
        _bn_relu_body,
        grid=(B, N // TN),
        in_specs=[
            pl.BlockSpec((1, TN, C_OUT), lambda b, i: (b, i, 0)),
            pl.BlockSpec((8, C_OUT), lambda b, i: (0, 0)),
            pl.BlockSpec((1, C_OUT), lambda b, i: (0, 0)),
            pl.BlockSpec((1, C_OUT), lambda b, i: (0, 0)),
        ],
        out_specs=pl.BlockSpec((1, C_OUT, TN), lambda b, i: (b, 0, i)),
        out_shape=jax.ShapeDtypeStruct((B, C_OUT, N), jnp.float32),
    )(y, sums, bn_gamma[None, :], bn_beta[None, :])

    return out


# speedup vs baseline: 1.1133x; 1.1133x over previous
import functools

import jax
import jax.numpy as jnp
from jax import lax
from jax.experimental import pallas as pl
from jax.experimental.pallas import tpu as pltpu
from jax.experimental.pallas import tpu_sc as plsc

B, C_IN, C_OUT, N, K = 4, 64, 128, 2048, 16
TN = 256
NEG_BIG = 3.0e38

NC, NS, L = 2, 16, 16
NW = NC * NS
PPW = (B * N) // NW
PC = 16
NCH = PPW // PC
ROWS = PC * K


def _knn_idx_body(rows_ref, full_ref, idx_ref):
    b = pl.program_id(0)
    i = pl.program_id(1)
    rows = rows_ref[0]
    full = full_ref[0]
    rn_rows = jnp.sum(rows * rows, axis=1, keepdims=True)
    rn_all = jnp.sum(full * full, axis=1)[None, :]
    cross = lax.dot_general(
        rows, full, dimension_numbers=(((1,), (1,)), ((), ())),
        preferred_element_type=jnp.float32)
    d = rn_rows + rn_all - 2.0 * cross

    col_i = lax.broadcasted_iota(jnp.int32, (TN, N), 1)
    row_i = lax.broadcasted_iota(jnp.int32, (TN, N), 0)
    d = jnp.where(col_i == row_i + i * TN, NEG_BIG, d)
    iota_f = col_i.astype(jnp.float32)
    self_col = (row_i[:, 0] + i * TN).astype(jnp.float32)
    cols = [self_col]
    for _ in range(K - 1):
        v = jnp.min(d, axis=1, keepdims=True)
        eq = d == v
        idxj = jnp.min(jnp.where(eq, iota_f, NEG_BIG), axis=1)
        cols.append(idxj)
        d = jnp.where(eq, NEG_BIG, d)
    idx_ref[...] = jnp.stack(cols, axis=1).astype(jnp.int32) + b * N


def _gather_max_body(idx_hbm, xt_hbm, out_hbm, idx_v, rows_v, xm_v, sem0, sem1):
    wid = lax.axis_index("s") * NC + lax.axis_index("c")
    base = wid * PPW
    pltpu.sync_copy(idx_hbm.at[pl.ds(base * K, PPW * K)], idx_v)

    sems = (sem0, sem1)
    pending = pltpu.async_copy(
        xt_hbm.at[idx_v.at[pl.ds(0, ROWS)]], rows_v.at[0], sems[0])
    for c in range(NCH):
        cur = pending
        if c + 1 < NCH:
            pending = pltpu.async_copy(
                xt_hbm.at[idx_v.at[pl.ds((c + 1) * ROWS, ROWS)]],
                rows_v.at[(c + 1) % 2], sems[(c + 1) % 2])
        cur.wait()
        buf = c % 2

        def point_body(p, carry, _c=c, _buf=buf):
            for ch in range(C_IN // L):
                sl = pl.ds(ch * L, L)
                acc = rows_v[_buf, p * K, sl]
                for j in range(1, K):
                    acc = jnp.maximum(acc, rows_v[_buf, p * K + j, sl])
                xm_v[_c * PC + p, sl] = acc
            return carry

        lax.fori_loop(0, PC, point_body, 0)

    pltpu.sync_copy(xm_v, out_hbm.at[pl.ds(base, PPW)])


_gather_max = functools.partial(
    pl.kernel,
    out_type=jax.ShapeDtypeStruct((B * N, C_IN), jnp.float32),
    mesh=plsc.VectorSubcoreMesh(core_axis_name="c", subcore_axis_name="s"),
    scratch_types=[
        pltpu.VMEM((PPW * K,), jnp.int32),
        pltpu.VMEM((2, ROWS, 2 * C_IN), jnp.float32),
        pltpu.VMEM((PPW, C_IN), jnp.float32),
        pltpu.SemaphoreType.DMA,
        pltpu.SemaphoreType.DMA,
    ],
)(_gather_max_body)


def _conv_stats_body(xm_ref, w_ref, b_ref, y_ref, sums_ref):
    step = pl.program_id(0) * pl.num_programs(1) + pl.program_id(1)
    xm = xm_ref[0]
    w = w_ref[...]
    y = lax.dot_general(
        xm, w, dimension_numbers=(((1,), (1,)), ((), ())),
        preferred_element_type=jnp.float32) + b_ref[...]
    y_ref[0] = y

    @pl.when(step == 0)
    def _():
        sums_ref[...] = jnp.zeros_like(sums_ref)

    sums_ref[0:1, :] += jnp.sum(y, axis=0, keepdims=True)
    sums_ref[1:2, :] += jnp.sum(y * y, axis=0, keepdims=True)


def _bn_relu_body(y_ref, sums_ref, g_ref, bt_ref, out_ref):
    y = y_ref[0]
    cnt = float(B * N)
    mean = sums_ref[0:1, :] / cnt
    var = sums_ref[1:2, :] / cnt - mean * mean
    scale = g_ref[...] / jnp.sqrt(var + 1e-5)
    shift = bt_ref[...] - mean * scale
    r = jnp.maximum(y * scale + shift, 0.0)
    out_ref[0] = r.T


def kernel(x, conv_w, conv_b, bn_gamma, bn_beta):
    xt = jnp.transpose(x, (0, 2, 1))
    w = conv_w[:, :, 0]

    idx = pl.pallas_call(
        _knn_idx_body,
        grid=(B, N // TN),
        in_specs=[
            pl.BlockSpec((1, TN, C_IN), lambda b, i: (b, i, 0)),
            pl.BlockSpec((1, N, C_IN), lambda b, i: (b, 0, 0)),
        ],
        out_specs=pl.BlockSpec((TN, K), lambda b, i: (b * (N // TN) + i, 0)),
        out_shape=jax.ShapeDtypeStruct((B * N, K), jnp.int32),
    )(xt, xt)

    xt_pad = jnp.pad(xt.reshape(B * N, C_IN), ((0, 0), (0, C_IN)))
    xm = _gather_max(idx.reshape(B * N * K), xt_pad)
    xm = xm.reshape(B, N, C_IN)

    y, sums = pl.pallas_call(
        _conv_stats_body,
        grid=(B, N // TN),
        in_specs=[
            pl.BlockSpec((1, TN, C_IN), lambda b, i: (b, i, 0)),
            pl.BlockSpec((C_OUT, C_IN), lambda b, i: (0, 0)),
            pl.BlockSpec((1, C_OUT), lambda b, i: (0, 0)),
        ],
        out_specs=[
            pl.BlockSpec((1, TN, C_OUT), lambda b, i: (b, i, 0)),
            pl.BlockSpec((8, C_OUT), lambda b, i: (0, 0)),
        ],
        out_shape=[
            jax.ShapeDtypeStruct((B, N, C_OUT), jnp.float32),
            jax.ShapeDtypeStruct((8, C_OUT), jnp.float32),
        ],
    )(xm, w, conv_b[None, :])

    out = pl.pallas_call(
        _bn_relu_body,
        grid=(B, N // TN),
        in_specs=[
            pl.BlockSpec((1, TN, C_OUT), lambda b, i: (b, i, 0)),
            pl.BlockSpec((8, C_OUT), lambda b, i: (0, 0)),
            pl.BlockSpec((1, C_OUT), lambda b, i: (0, 0)),
            pl.BlockSpec((1, C_OUT), lambda b, i: (0, 0)),
        ],
        out_specs=pl.BlockSpec((1, C_OUT, TN), lambda b, i: (b, 0, i)),
        out_shape=jax.ShapeDtypeStruct((B, C_OUT, N), jnp.float32),
    )(y, sums, bn_gamma[None, :], bn_beta[None, :])

    return out


# speedup vs baseline: 1.2185x; 1.0945x over previous
import functools

import jax
import jax.numpy as jnp
from jax import lax
from jax.experimental import pallas as pl
from jax.experimental.pallas import tpu as pltpu
from jax.experimental.pallas import tpu_sc as plsc

B, C_IN, C_OUT, N, K = 4, 64, 128, 2048, 16
TN = 256
NT = N // TN
NEG_BIG = 3.0e38

NC, NS, L = 2, 16, 16
NW = NC * NS
PPW = (B * N) // NW
PC = 32
NCH = PPW // PC
ROWS = PC * K


def _knn_idx_body(rowsx_ref, fullx_ref, idx_ref, tab_ref):
    b = pl.program_id(0)
    i = pl.program_id(1)
    rows_x = rowsx_ref[0]
    full_x = fullx_ref[0]
    xt_tile = rows_x.T
    tab_ref[...] = xt_tile
    rn_rows = jnp.sum(xt_tile * xt_tile, axis=1, keepdims=True)
    rn_all = jnp.sum(full_x * full_x, axis=0)[None, :]
    cross = lax.dot_general(
        rows_x, full_x, dimension_numbers=(((0,), (0,)), ((), ())),
        preferred_element_type=jnp.float32)
    d = rn_rows + rn_all - 2.0 * cross

    col_i = lax.broadcasted_iota(jnp.int32, (TN, N), 1)
    row_i = lax.broadcasted_iota(jnp.int32, (TN, N), 0)
    d = jnp.where(col_i == row_i + i * TN, NEG_BIG, d)
    iota_f = col_i.astype(jnp.float32)
    self_col = (row_i[:, 0] + i * TN).astype(jnp.float32)
    cols = [self_col]
    for _ in range(K - 1):
        v = jnp.min(d, axis=1, keepdims=True)
        eq = d == v
        idxj = jnp.min(jnp.where(eq, iota_f, NEG_BIG), axis=1)
        cols.append(idxj)
        d = jnp.where(eq, NEG_BIG, d)
    idx_ref[...] = jnp.stack(cols, axis=1).astype(jnp.int32) + b * N


def _gather_max_body(idx_hbm, xt_hbm, out_hbm, idx_v, rows_v, xm_v, sem0, sem1):
    wid = lax.axis_index("s") * NC + lax.axis_index("c")
    base = wid * PPW
    pltpu.sync_copy(idx_hbm.at[pl.ds(base * K, PPW * K)], idx_v)

    sems = (sem0, sem1)
    pending = pltpu.async_copy(
        xt_hbm.at[idx_v.at[pl.ds(0, ROWS)]], rows_v.at[0], sems[0])
    for c in range(NCH):
        cur = pending
        if c + 1 < NCH:
            pending = pltpu.async_copy(
                xt_hbm.at[idx_v.at[pl.ds((c + 1) * ROWS, ROWS)]],
                rows_v.at[(c + 1) % 2], sems[(c + 1) % 2])
        cur.wait()
        buf = c % 2

        def point_body(p, carry, _c=c, _buf=buf):
            for ch in range(C_IN // L):
                sl = pl.ds(ch * L, L)
                acc = rows_v[_buf, p * K, sl]
                for j in range(1, K):
                    acc = jnp.maximum(acc, rows_v[_buf, p * K + j, sl])
                xm_v[_c * PC + p, sl] = acc
            return carry

        lax.fori_loop(0, PC, point_body, 0)

    pltpu.sync_copy(xm_v, out_hbm.at[pl.ds(base, PPW)])


_gather_max = functools.partial(
    pl.kernel,
    out_type=jax.ShapeDtypeStruct((B * N, C_IN), jnp.float32),
    mesh=plsc.VectorSubcoreMesh(core_axis_name="c", subcore_axis_name="s"),
    compiler_params=pltpu.CompilerParams(use_tc_tiling_on_sc=False),
    scratch_types=[
        pltpu.VMEM((PPW * K,), jnp.int32),
        pltpu.VMEM((2, ROWS, C_IN), jnp.float32),
        pltpu.VMEM((PPW, C_IN), jnp.float32),
        pltpu.SemaphoreType.DMA,
        pltpu.SemaphoreType.DMA,
    ],
)(_gather_max_body)


def _conv_bn_body(xm_ref, w_ref, b_ref, g_ref, bt_ref, out_ref, y_all, sums_ref):
    ph = pl.program_id(0)
    step = pl.program_id(1) * pl.num_programs(2) + pl.program_id(2)
    row0 = step * TN

    @pl.when(jnp.logical_and(ph == 0, step == 0))
    def _():
        sums_ref[...] = jnp.zeros_like(sums_ref)

    @pl.when(ph == 0)
    def _():
        y = lax.dot_general(
            xm_ref[0], w_ref[...], dimension_numbers=(((1,), (1,)), ((), ())),
            preferred_element_type=jnp.float32) + b_ref[...]
        y_all[pl.ds(row0, TN), :] = y
        sums_ref[0:1, :] += jnp.sum(y, axis=0, keepdims=True)
        sums_ref[1:2, :] += jnp.sum(y * y, axis=0, keepdims=True)

    @pl.when(ph == 1)
    def _():
        cnt = float(B * N)
        mean = sums_ref[0:1, :] / cnt
        var = sums_ref[1:2, :] / cnt - mean * mean
        scale = g_ref[...] / jnp.sqrt(var + 1e-5)
        shift = bt_ref[...] - mean * scale
        y = y_all[pl.ds(row0, TN), :]
        r = jnp.maximum(y * scale + shift, 0.0)
        out_ref[0] = r.T


def kernel(x, conv_w, conv_b, bn_gamma, bn_beta):
    w = conv_w[:, :, 0]

    idx, tab = pl.pallas_call(
        _knn_idx_body,
        grid=(B, NT),
        in_specs=[
            pl.BlockSpec((1, C_IN, TN), lambda b, i: (b, 0, i)),
            pl.BlockSpec((1, C_IN, N), lambda b, i: (b, 0, 0)),
        ],
        out_specs=[
            pl.BlockSpec((TN, K), lambda b, i: (b * NT + i, 0)),
            pl.BlockSpec((TN, C_IN), lambda b, i: (b * NT + i, 0)),
        ],
        out_shape=[
            jax.ShapeDtypeStruct((B * N, K), jnp.int32),
            jax.ShapeDtypeStruct((B * N, C_IN), jnp.float32),
        ],
    )(x, x)

    xm = _gather_max(idx.reshape(B * N * K), tab)
    xm = xm.reshape(B, N, C_IN)

    out = pl.pallas_call(
        _conv_bn_body,
        grid=(2, B, NT),
        in_specs=[
            pl.BlockSpec((1, TN, C_IN), lambda p, b, i: (b, i, 0)),
            pl.BlockSpec((C_OUT, C_IN), lambda p, b, i: (0, 0)),
            pl.BlockSpec((1, C_OUT), lambda p, b, i: (0, 0)),
            pl.BlockSpec((1, C_OUT), lambda p, b, i: (0, 0)),
            pl.BlockSpec((1, C_OUT), lambda p, b, i: (0, 0)),
        ],
        out_specs=pl.BlockSpec((1, C_OUT, TN), lambda p, b, i: (b, 0, i)),
        out_shape=jax.ShapeDtypeStruct((B, C_OUT, N), jnp.float32),
        scratch_shapes=[
            pltpu.VMEM((B * N, C_OUT), jnp.float32),
            pltpu.VMEM((8, C_OUT), jnp.float32),
        ],
    )(xm, w, conv_b[None, :], bn_gamma[None, :], bn_beta[None, :])

    return out


# speedup vs baseline: 1.2920x; 1.0603x over previous
import functools

import jax
import jax.numpy as jnp
from jax import lax
from jax.experimental import pallas as pl
from jax.experimental.pallas import tpu as pltpu
from jax.experimental.pallas import tpu_sc as plsc

B, C_IN, C_OUT, N, K = 4, 64, 128, 2048, 16
TN = 512
NT = N // TN
NEG_BIG = 3.0e38

NC, NS, L = 2, 16, 16
NW = NC * NS
PPW = (B * N) // NW
PC = 32
NCH = PPW // PC
ROWS = PC * K


def _knn_idx_body(rowsx_ref, fullx_ref, idx_ref, tab_ref):
    b = pl.program_id(0)
    i = pl.program_id(1)
    rows_x = rowsx_ref[0]
    full_x = fullx_ref[0]
    xt_tile = rows_x.T
    tab_ref[...] = xt_tile
    rn_rows = jnp.sum(xt_tile * xt_tile, axis=1, keepdims=True)
    rn_all = jnp.sum(full_x * full_x, axis=0)[None, :]
    cross = lax.dot_general(
        rows_x, full_x, dimension_numbers=(((0,), (0,)), ((), ())),
        preferred_element_type=jnp.float32)
    d = rn_rows + rn_all - 2.0 * cross

    col_i = lax.broadcasted_iota(jnp.int32, (TN, N), 1)
    row_i = lax.broadcasted_iota(jnp.int32, (TN, N), 0)
    d = jnp.where(col_i == row_i + i * TN, NEG_BIG, d)
    iota_f = col_i.astype(jnp.float32)
    self_col = (row_i[:, 0] + i * TN).astype(jnp.float32)
    cols = [self_col]
    for _ in range(K - 1):
        v = jnp.min(d, axis=1, keepdims=True)
        eq = d == v
        idxj = jnp.min(jnp.where(eq, iota_f, NEG_BIG), axis=1)
        cols.append(idxj)
        d = jnp.where(eq, NEG_BIG, d)
    idx_ref[...] = jnp.stack(cols, axis=1).astype(jnp.int32) + b * N


def _gather_max_body(idx_hbm, xt_hbm, out_hbm, idx_v, rows_v, xm_v, sem0, sem1):
    wid = lax.axis_index("s") * NC + lax.axis_index("c")
    base = wid * PPW
    pltpu.sync_copy(idx_hbm.at[pl.ds(base * K, PPW * K)], idx_v)

    sems = (sem0, sem1)
    pending = pltpu.async_copy(
        xt_hbm.at[idx_v.at[pl.ds(0, ROWS)]], rows_v.at[0], sems[0])
    for c in range(NCH):
        cur = pending
        if c + 1 < NCH:
            pending = pltpu.async_copy(
                xt_hbm.at[idx_v.at[pl.ds((c + 1) * ROWS, ROWS)]],
                rows_v.at[(c + 1) % 2], sems[(c + 1) % 2])
        cur.wait()
        buf = c % 2

        def point_body(p, carry, _c=c, _buf=buf):
            for ch in range(C_IN // L):
                sl = pl.ds(ch * L, L)
                acc = rows_v[_buf, p * K, sl]
                for j in range(1, K):
                    acc = jnp.maximum(acc, rows_v[_buf, p * K + j, sl])
                xm_v[_c * PC + p, sl] = acc
            return carry

        lax.fori_loop(0, PC, point_body, 0)

    pltpu.sync_copy(xm_v, out_hbm.at[pl.ds(base, PPW)])


_gather_max = functools.partial(
    pl.kernel,
    out_type=jax.ShapeDtypeStruct((B * N, C_IN), jnp.float32),
    mesh=plsc.VectorSubcoreMesh(core_axis_name="c", subcore_axis_name="s"),
    compiler_params=pltpu.CompilerParams(use_tc_tiling_on_sc=False),
    scratch_types=[
        pltpu.VMEM((PPW * K,), jnp.int32),
        pltpu.VMEM((2, ROWS, C_IN), jnp.float32),
        pltpu.VMEM((PPW, C_IN), jnp.float32),
        pltpu.SemaphoreType.DMA,
        pltpu.SemaphoreType.DMA,
    ],
)(_gather_max_body)


def _conv_bn_body(xm_ref, w_ref, b_ref, g_ref, bt_ref, out_ref, y_all, sums_ref):
    ph = pl.program_id(0)
    step = pl.program_id(1) * pl.num_programs(2) + pl.program_id(2)
    row0 = step * TN

    @pl.when(jnp.logical_and(ph == 0, step == 0))
    def _():
        sums_ref[...] = jnp.zeros_like(sums_ref)

    @pl.when(ph == 0)
    def _():
        y = lax.dot_general(
            xm_ref[0], w_ref[...], dimension_numbers=(((1,), (1,)), ((), ())),
            preferred_element_type=jnp.float32) + b_ref[...]
        y_all[pl.ds(row0, TN), :] = y
        sums_ref[0:1, :] += jnp.sum(y, axis=0, keepdims=True)
        sums_ref[1:2, :] += jnp.sum(y * y, axis=0, keepdims=True)

    @pl.when(ph == 1)
    def _():
        cnt = float(B * N)
        mean = sums_ref[0:1, :] / cnt
        var = sums_ref[1:2, :] / cnt - mean * mean
        scale = g_ref[...] / jnp.sqrt(var + 1e-5)
        shift = bt_ref[...] - mean * scale
        y = y_all[pl.ds(row0, TN), :]
        r = jnp.maximum(y * scale + shift, 0.0)
        out_ref[0] = r.T


def kernel(x, conv_w, conv_b, bn_gamma, bn_beta):
    w = conv_w[:, :, 0]

    idx, tab = pl.pallas_call(
        _knn_idx_body,
        grid=(B, NT),
        in_specs=[
            pl.BlockSpec((1, C_IN, TN), lambda b, i: (b, 0, i)),
            pl.BlockSpec((1, C_IN, N), lambda b, i: (b, 0, 0)),
        ],
        out_specs=[
            pl.BlockSpec((TN, K), lambda b, i: (b * NT + i, 0)),
            pl.BlockSpec((TN, C_IN), lambda b, i: (b * NT + i, 0)),
        ],
        out_shape=[
            jax.ShapeDtypeStruct((B * N, K), jnp.int32),
            jax.ShapeDtypeStruct((B * N, C_IN), jnp.float32),
        ],
    )(x, x)

    xm = _gather_max(idx.reshape(B * N * K), tab)
    xm = xm.reshape(B, N, C_IN)

    out = pl.pallas_call(
        _conv_bn_body,
        grid=(2, B, NT),
        in_specs=[
            pl.BlockSpec((1, TN, C_IN), lambda p, b, i: (b, i, 0)),
            pl.BlockSpec((C_OUT, C_IN), lambda p, b, i: (0, 0)),
            pl.BlockSpec((1, C_OUT), lambda p, b, i: (0, 0)),
            pl.BlockSpec((1, C_OUT), lambda p, b, i: (0, 0)),
            pl.BlockSpec((1, C_OUT), lambda p, b, i: (0, 0)),
        ],
        out_specs=pl.BlockSpec((1, C_OUT, TN), lambda p, b, i: (b, 0, i)),
        out_shape=jax.ShapeDtypeStruct((B, C_OUT, N), jnp.float32),
        scratch_shapes=[
            pltpu.VMEM((B * N, C_OUT), jnp.float32),
            pltpu.VMEM((8, C_OUT), jnp.float32),
        ],
    )(xm, w, conv_b[None, :], bn_gamma[None, :], bn_beta[None, :])

    return out
